# baseline (speedup 1.0000x reference)
import functools
import jax
import jax.numpy as jnp
from jax.experimental import pallas as pl
from jax.experimental.pallas import tpu as pltpu


def _chamfer_kernel(src_ref, tgt_ref, out_src_ref, out_dst_ref, *, blocks):
    def body(b, acc):
        src_acc, dst_acc = acc
        n = src_ref.shape[0] // blocks
        m = tgt_ref.shape[0] // blocks
        s = src_ref[pl.ds(b * n, n), :]
        t = tgt_ref[pl.ds(b * m, m), :]
        tt = t.T

        sx = s[:, 0:1]
        sy = s[:, 1:2]
        sz = s[:, 2:3]
        tx = tt[0:1, :]
        ty = tt[1:2, :]
        tz = tt[2:3, :]

        dx = sx - tx
        dy = sy - ty
        dz = sz - tz
        dist = dx * dx + dy * dy + dz * dz

        row_min = jnp.min(dist, axis=1, keepdims=True)
        col_min = jnp.min(dist, axis=0, keepdims=True)

        src_acc += jnp.sum(row_min, axis=0, keepdims=True)
        dst_acc += jnp.sum(col_min, axis=1, keepdims=True)
        return src_acc, dst_acc

    zero = jnp.zeros((1, 1), jnp.float32)
    src_acc, dst_acc = jax.lax.fori_loop(0, blocks, body, (zero, zero))
    out_src_ref[...] = src_acc * (1.0 / src_ref.shape[0])
    out_dst_ref[...] = dst_acc * (1.0 / tgt_ref.shape[0])


def kernel(source, target):
    B, N, C = source.shape
    M = target.shape[1]

    src_flat = source.reshape(B * N, C)
    tgt_flat = target.reshape(B * M, C)

    out_src, out_dst = pl.pallas_call(
        functools.partial(_chamfer_kernel, blocks=B),
        out_shape=[
            jax.ShapeDtypeStruct((1, 1), jnp.float32),
            jax.ShapeDtypeStruct((1, 1), jnp.float32),
        ],
    )(src_flat, tgt_flat)

    return (out_src[0, 0], out_dst[0, 0])
